# baseline (speedup 1.0000x reference)
import jax
import jax.numpy as jnp
from jax.experimental import pallas as pl
from jax.experimental.pallas import tpu as pltpu

HEADS = 4
THETA = 0.7
VTH = 1.0
TOPK = 4
TAU = 2.0

T, B, C, D, H, W = 4, 2, 96, 8, 28, 28
N = T * B
NW = 8
LD, LH, LW = 4, 14, 14
WIN = LD * LH * LW
CP = 128
NWC = NW * CP
S = D * H * W
EPS = 1e-5
HALF = WIN // 2


def _lif_step(vref, x, first):
    v_prev = jnp.where(first, jnp.zeros_like(x), vref[...])
    v = v_prev + (x - v_prev) / TAU
    s = (v >= VTH).astype(jnp.float32)
    vref[...] = v * (1.0 - s)
    return s


def _k1_body(xp_ref, wq_ref, wk_ref, qw_ref, kw_ref, xw_ref, st_ref, reg_ref):
    g = pl.program_id(1)

    def win_split(slab):
        u = slab.reshape(2, LH, 2, LW, C)
        u = jnp.transpose(u, (0, 2, 1, 3, 4)).reshape(4, LH * LW, C)
        return jnp.pad(u, ((0, 0), (0, 0), (0, CP - C)))

    WP = W + 4

    def conv_slab(d):
        accs = [jnp.zeros((H * WP, C), jnp.float32) for _ in range(3)]
        for dz in range(3):
            for dy in range(3):
                strip = xp_ref[0, d + dz, dy:dy + H, :, :].reshape(H * WP, C)
                for dx in range(3):
                    tap = dz * 9 + dy * 3 + dx
                    accs[dx] = accs[dx] + jnp.dot(
                        strip, wq_ref[tap], preferred_element_type=jnp.float32)
        a0 = accs[0].reshape(H, WP, C)
        a1 = accs[1].reshape(H, WP, C)
        a2 = accs[2].reshape(H, WP, C)
        return a0[:, 0:W, :] + a1[:, 1:W + 1, :] + a2[:, 2:W + 2, :]

    stat = [jnp.zeros((C,), jnp.float32) for _ in range(6)]
    regq = [jnp.zeros((4, C), jnp.float32) for _ in range(2)]
    regk = [jnp.zeros((4, C), jnp.float32) for _ in range(2)]
    for li in range(2):
        r0, r1 = li * 196, (li + 1) * 196
        for zt in range(2):
            d = zt * LD + 2 * g + li
            qs = conv_slab(d)
            cfull = xp_ref[0, d + 1, 1:1 + H, :, :]
            ksf = jnp.dot(cfull.reshape(H * (W + 4), C), wk_ref[...],
                          preferred_element_type=jnp.float32
                          ).reshape(H, W + 4, C)
            center = cfull[:, 1:1 + W, :]
            ks = ksf[:, 1:1 + W, :]
            q4 = win_split(qs)
            k4 = win_split(ks)
            x4 = win_split(center)
            for q in range(4):
                c0 = (zt * 4 + q) * CP
                qw_ref[0, 0, r0:r1, c0:c0 + CP] = q4[q]
                kw_ref[0, 0, r0:r1, c0:c0 + CP] = k4[q]
                xw_ref[0, 0, r0:r1, c0:c0 + CP] = x4[q]
            q4 = q4[:, :, :C]
            k4 = k4[:, :, :C]
            x4 = x4[:, :, :C]
            regq[zt] = regq[zt] + jnp.sum(q4, axis=1)
            regk[zt] = regk[zt] + jnp.sum(k4, axis=1)
            stat[0] = stat[0] + jnp.sum(q4, axis=(0, 1))
            stat[1] = stat[1] + jnp.sum(q4 * q4, axis=(0, 1))
            stat[2] = stat[2] + jnp.sum(k4, axis=(0, 1))
            stat[3] = stat[3] + jnp.sum(k4 * k4, axis=(0, 1))
            stat[4] = stat[4] + jnp.sum(x4, axis=(0, 1))
            stat[5] = stat[5] + jnp.sum(x4 * x4, axis=(0, 1))

    for i in range(6):
        st_ref[0, 0, i, :] = stat[i]
    reg_ref[0, 0, 0:4, :] = regq[0]
    reg_ref[0, 0, 4:8, :] = regq[1]
    reg_ref[0, 0, 8:12, :] = regk[0]
    reg_ref[0, 0, 12:16, :] = regk[1]


def _run_k1(xp, wq, wk):
    win_spec = pl.BlockSpec((1, 1, HALF, NWC),
                            lambda n, g: (n // B, n % B, g, 0))
    out_shapes = (
        jax.ShapeDtypeStruct((T, B, WIN, NWC), jnp.float32),
        jax.ShapeDtypeStruct((T, B, WIN, NWC), jnp.float32),
        jax.ShapeDtypeStruct((T, B, WIN, NWC), jnp.float32),
        jax.ShapeDtypeStruct((N, 2, 6, C), jnp.float32),
        jax.ShapeDtypeStruct((N, 2, 16, C), jnp.float32),
    )
    return pl.pallas_call(
        _k1_body,
        grid=(N, 2),
        in_specs=[
            pl.BlockSpec((1, D + 2, H + 2, W + 4, C),
                         lambda n, g: (n, 0, 0, 0, 0)),
            pl.BlockSpec((27, C, C), lambda n, g: (0, 0, 0)),
            pl.BlockSpec((C, C), lambda n, g: (0, 0)),
        ],
        out_specs=[win_spec, win_spec, win_spec,
                   pl.BlockSpec((1, 1, 6, C), lambda n, g: (n, g, 0, 0)),
                   pl.BlockSpec((1, 1, 16, C), lambda n, g: (n, g, 0, 0))],
        out_shape=out_shapes,
    )(xp, wq, wk)


def _k2_body(reg_ref, aff_ref, r_ref):
    inv = 1.0 / (T * WIN)
    for b in range(B):
        qreg = reg_ref[b, 0, :, :] * inv * aff_ref[0, :] + aff_ref[1, :]
        kreg = reg_ref[b, 1, :, :] * inv * aff_ref[2, :] + aff_ref[3, :]
        a = jnp.dot(qreg, kreg.T, preferred_element_type=jnp.float32)
        col = jax.lax.broadcasted_iota(jnp.int32, (NW, NW), 1)
        elig = jnp.ones((NW, NW), jnp.bool_)
        sel = jnp.zeros((NW, NW), jnp.float32)
        for _ in range(TOPK):
            masked = jnp.where(elig, a, -jnp.inf)
            m = jnp.max(masked, axis=1, keepdims=True)
            ism = masked >= m
            first_col = jnp.min(jnp.where(ism, col, NW), axis=1, keepdims=True)
            pick = col == first_col
            sel = sel + jnp.where(pick, 1.0 / TOPK, 0.0)
            elig = jnp.logical_and(elig, jnp.logical_not(pick))
        r_ref[b, :, :] = sel


def _run_k2(regsum, aff):
    return pl.pallas_call(
        _k2_body,
        out_shape=jax.ShapeDtypeStruct((B, NW, NW), jnp.float32),
    )(regsum, aff)


def _k3_body(qw_ref, kw_ref, xw_ref, mr_ref, aff_ref, wp_ref, seg_ref,
             segt_ref, yw_ref, st_ref, vq_ref, vk_ref, vv_ref, va_ref,
             vo_ref):
    t = pl.program_id(1)
    first = t == 0

    q_in = qw_ref[0, 0] * aff_ref[0, :] + aff_ref[1, :]
    k_in = kw_ref[0, 0] * aff_ref[2, :] + aff_ref[3, :]
    v_in = xw_ref[0, 0] * aff_ref[4, :] + aff_ref[5, :]

    qs = _lif_step(vq_ref, q_in, first)
    ks = _lif_step(vk_ref, k_in, first)
    vs = _lif_step(vv_ref, v_in, first)

    mroute = mr_ref[0]
    k_agg = jnp.dot(ks, mroute, preferred_element_type=jnp.float32)
    v_agg = jnp.dot(vs, mroute, preferred_element_type=jnp.float32)

    attn_pre = jnp.dot(qs * k_agg, seg_ref[...],
                       preferred_element_type=jnp.float32)
    a_s = _lif_step(va_ref, attn_pre, first)
    a_exp = jnp.dot(a_s, segt_ref[...], preferred_element_type=jnp.float32)

    os_ = _lif_step(vo_ref, a_exp * v_agg, first)

    y = jnp.dot(os_, wp_ref[...], preferred_element_type=jnp.float32)
    yw_ref[0, 0] = y
    st_ref[0, 0, 0, :] = jnp.sum(y, axis=0)
    st_ref[0, 0, 1, :] = jnp.sum(y * y, axis=0)


def _run_k3(qw, kw, xw, mroute, aff, wp768, seg768, seg768t):
    full = pl.BlockSpec((1, 1, WIN, NWC), lambda b, t: (t, b, 0, 0))
    return pl.pallas_call(
        _k3_body,
        grid=(B, T),
        in_specs=[full, full, full,
                  pl.BlockSpec((1, NWC, NWC), lambda b, t: (b, 0, 0)),
                  pl.BlockSpec((6, NWC), lambda b, t: (0, 0)),
                  pl.BlockSpec((NWC, NWC), lambda b, t: (0, 0)),
                  pl.BlockSpec((NWC, 4 * NW), lambda b, t: (0, 0)),
                  pl.BlockSpec((4 * NW, NWC), lambda b, t: (0, 0))],
        out_specs=[full,
                   pl.BlockSpec((1, 1, 2, NWC), lambda b, t: (b, t, 0, 0))],
        out_shape=(jax.ShapeDtypeStruct((T, B, WIN, NWC), jnp.float32),
                   jax.ShapeDtypeStruct((B, T, 2, NWC), jnp.float32)),
        scratch_shapes=[
            pltpu.VMEM((WIN, NWC), jnp.float32),
            pltpu.VMEM((WIN, NWC), jnp.float32),
            pltpu.VMEM((WIN, NWC), jnp.float32),
            pltpu.VMEM((WIN, 4 * NW), jnp.float32),
            pltpu.VMEM((WIN, NWC), jnp.float32),
        ],
    )(qw, kw, xw, mroute, aff, wp768, seg768, seg768t)


def _k4_body(yw_ref, aff_ref, out_ref):
    out_ref[0, 0] = yw_ref[0, 0] * aff_ref[0, :] + aff_ref[1, :]


def _run_k4(yw, aff):
    spec = pl.BlockSpec((1, 1, WIN, NWC), lambda n: (n // B, n % B, 0, 0))
    return pl.pallas_call(
        _k4_body,
        grid=(N,),
        in_specs=[spec, pl.BlockSpec((2, NWC), lambda n: (0, 0))],
        out_specs=spec,
        out_shape=jax.ShapeDtypeStruct((T, B, WIN, NWC), jnp.float32),
    )(yw, aff)


def _affine(g, b, ssum, ssq, count):
    mean = ssum / count
    var = ssq / count - mean * mean
    scale = g * jax.lax.rsqrt(var + EPS)
    return scale, b - mean * scale


@jax.jit
def kernel(x, Wq, gq, bq, Wk, gk, bk, gv, bv, Wp, gp, bp):
    kd = Wq[:, :, 0].sum(axis=(2, 3)) + Wq[:, :, 2].sum(axis=(2, 3))
    w_eff = Wq.at[:, :, 1, 1, 1].add(-THETA * kd)
    wq_t = jnp.transpose(w_eff, (2, 3, 4, 1, 0)).reshape(27, C, C)
    wk_t = Wk[:, :, 0, 0, 0].T
    eye8 = jnp.eye(NW, dtype=jnp.float32)
    wp_pad = jnp.pad(Wp[:, :, 0, 0, 0].T, ((0, CP - C), (0, CP - C)))
    wp768 = jnp.kron(eye8, wp_pad)
    seg96 = (jax.lax.broadcasted_iota(jnp.int32, (C, HEADS), 0) // (C // HEADS)
             == jax.lax.broadcasted_iota(jnp.int32, (C, HEADS), 1)
             ).astype(jnp.float32)
    seg768 = jnp.kron(eye8, jnp.pad(seg96, ((0, CP - C), (0, 0))))
    seg768t = seg768.T

    xl = jnp.transpose(x.reshape(N, C, D, H, W), (0, 2, 3, 4, 1))
    xp = jnp.pad(xl, ((0, 0), (1, 1), (1, 1), (1, 3), (0, 0)))

    qw, kw, xw, st, reg = _run_k1(xp, wq_t, wk_t)

    stt = st.sum(axis=(0, 1))
    cnt = float(N * S)
    qsc, qsh = _affine(gq, bq, stt[0], stt[1], cnt)
    ksc, ksh = _affine(gk, bk, stt[2], stt[3], cnt)
    vsc, vsh = _affine(gv, bv, stt[4], stt[5], cnt)
    aff2 = jnp.stack([qsc, qsh, ksc, ksh])
    aff6 = jnp.tile(jnp.pad(jnp.stack([qsc, qsh, ksc, ksh, vsc, vsh]),
                            ((0, 0), (0, CP - C))), (1, NW))

    regs = reg.reshape(T, B, 2, 2, 2, 4, C).sum(axis=(0, 2)).reshape(B, 2, NW, C)
    r = _run_k2(regs, aff2)
    mroute = jax.vmap(lambda rb: jnp.kron(rb.T, jnp.eye(CP, dtype=jnp.float32)))(r)

    yw, st3 = _run_k3(qw, kw, xw, mroute, aff6, wp768, seg768, seg768t)
    stt3 = st3.sum(axis=(0, 1)).reshape(2, NW, CP)[:, :, :C].sum(axis=1)
    psc, psh = _affine(gp, bp, stt3[0], stt3[1], cnt)
    outw = _run_k4(yw, jnp.tile(jnp.pad(jnp.stack([psc, psh]),
                                        ((0, 0), (0, CP - C))), (1, NW)))

    o = outw.reshape(T, B, LD, LH, LW, 2, 2, 2, CP)[..., :C]
    o = jnp.transpose(o, (0, 1, 8, 5, 2, 6, 3, 7, 4))
    return o.reshape(T, B, C, D, H, W)


# speedup vs baseline: 1.1661x; 1.1661x over previous
import jax
import jax.numpy as jnp
from jax.experimental import pallas as pl
from jax.experimental.pallas import tpu as pltpu

HEADS = 4
THETA = 0.7
VTH = 1.0
TOPK = 4
TAU = 2.0

T, B, C, D, H, W = 4, 2, 96, 8, 28, 28
N = T * B
NW = 8
LD, LH, LW = 4, 14, 14
WIN = LD * LH * LW
S = D * H * W
EPS = 1e-5


def _lif_step(vref, x, first):
    v_prev = jnp.where(first, jnp.zeros_like(x), vref[...])
    v = v_prev + (x - v_prev) / TAU
    s = (v >= VTH).astype(jnp.float32)
    vref[...] = v * (1.0 - s)
    return s


def _k1_body(xp_ref, wq_ref, wk_ref, qw_ref, kw_ref, xw_ref, st_ref, reg_ref):
    d = pl.program_id(1)

    def win_split(t):
        t = t.reshape(2, LH, 2, LW, C)
        t = jnp.transpose(t, (0, 2, 1, 3, 4))
        return t.reshape(4, LH * LW, C)

    acc = jnp.zeros((H * W, C), jnp.float32)
    for tap in range(27):
        dz, rem = tap // 9, tap % 9
        dy, dx = rem // 3, rem % 3
        plane = xp_ref[0, d + dz, dy:dy + H, dx:dx + W, :].reshape(H * W, C)
        acc = acc + jnp.dot(plane, wq_ref[tap],
                            preferred_element_type=jnp.float32)
    center = xp_ref[0, d + 1, 1:1 + H, 1:1 + W, :].reshape(H * W, C)
    kout = jnp.dot(center, wk_ref[...], preferred_element_type=jnp.float32)

    qsplit = win_split(acc.reshape(H, W, C))
    ksplit = win_split(kout.reshape(H, W, C))
    xsplit = win_split(center.reshape(H, W, C))
    qw_ref[0, 0, :, 0, :, :] = qsplit
    kw_ref[0, 0, :, 0, :, :] = ksplit
    xw_ref[0, 0, :, 0, :, :] = xsplit

    st_ref[0, 0, 0, :] = jnp.sum(acc, axis=0)
    st_ref[0, 0, 1, :] = jnp.sum(acc * acc, axis=0)
    st_ref[0, 0, 2, :] = jnp.sum(kout, axis=0)
    st_ref[0, 0, 3, :] = jnp.sum(kout * kout, axis=0)
    st_ref[0, 0, 4, :] = jnp.sum(center, axis=0)
    st_ref[0, 0, 5, :] = jnp.sum(center * center, axis=0)

    reg_ref[0, 0, 0:4, :] = jnp.sum(qsplit, axis=1)
    reg_ref[0, 0, 4:8, :] = jnp.sum(ksplit, axis=1)


def _run_k1(xp, wq, wk):
    grid = (N, D)
    out_shapes = (
        jax.ShapeDtypeStruct((T, B, NW, LD, LH * LW, C), jnp.float32),
        jax.ShapeDtypeStruct((T, B, NW, LD, LH * LW, C), jnp.float32),
        jax.ShapeDtypeStruct((T, B, NW, LD, LH * LW, C), jnp.float32),
        jax.ShapeDtypeStruct((N, D, 6, C), jnp.float32),
        jax.ShapeDtypeStruct((N, D, 8, C), jnp.float32),
    )
    win_spec = pl.BlockSpec(
        (1, 1, 4, 1, LH * LW, C),
        lambda n, d: (n // B, n % B, d // LD, d % LD, 0, 0),
    )
    return pl.pallas_call(
        _k1_body,
        grid=grid,
        in_specs=[
            pl.BlockSpec((1, D + 2, H + 2, W + 2, C), lambda n, d: (n, 0, 0, 0, 0)),
            pl.BlockSpec((27, C, C), lambda n, d: (0, 0, 0)),
            pl.BlockSpec((C, C), lambda n, d: (0, 0)),
        ],
        out_specs=[win_spec, win_spec, win_spec,
                   pl.BlockSpec((1, 1, 6, C), lambda n, d: (n, d, 0, 0)),
                   pl.BlockSpec((1, 1, 8, C), lambda n, d: (n, d, 0, 0))],
        out_shape=out_shapes,
    )(xp, wq, wk)


def _k2_body(reg_ref, aff_ref, r_ref):
    inv = 1.0 / (T * WIN)
    for b in range(B):
        qreg = reg_ref[b, 0, :, :] * inv * aff_ref[0, :] + aff_ref[1, :]
        kreg = reg_ref[b, 1, :, :] * inv * aff_ref[2, :] + aff_ref[3, :]
        a = jnp.dot(qreg, kreg.T, preferred_element_type=jnp.float32)
        col = jax.lax.broadcasted_iota(jnp.int32, (NW, NW), 1)
        elig = jnp.ones((NW, NW), jnp.bool_)
        sel = jnp.zeros((NW, NW), jnp.float32)
        for _ in range(TOPK):
            masked = jnp.where(elig, a, -jnp.inf)
            m = jnp.max(masked, axis=1, keepdims=True)
            ism = masked >= m
            first_col = jnp.min(jnp.where(ism, col, NW), axis=1, keepdims=True)
            pick = col == first_col
            sel = sel + jnp.where(pick, 1.0 / TOPK, 0.0)
            elig = jnp.logical_and(elig, jnp.logical_not(pick))
        r_ref[b, :, :] = sel


def _run_k2(regsum, aff):
    return pl.pallas_call(
        _k2_body,
        out_shape=jax.ShapeDtypeStruct((B, NW, NW), jnp.float32),
    )(regsum, aff)


def _k3_body(qw_ref, kw_ref, xw_ref, r_ref, aff_ref, wp_ref, seg_ref,
             yw_ref, st_ref, vq_ref, vk_ref, vv_ref, va_ref, vo_ref):
    t = pl.program_id(1)
    first = t == 0

    q_in = qw_ref[0, 0].reshape(NW, WIN, C) * aff_ref[0, :] + aff_ref[1, :]
    k_in = kw_ref[0, 0].reshape(NW, WIN, C) * aff_ref[2, :] + aff_ref[3, :]
    v_in = xw_ref[0, 0].reshape(NW, WIN, C) * aff_ref[4, :] + aff_ref[5, :]

    qs = _lif_step(vq_ref, q_in, first)
    ks = _lif_step(vk_ref, k_in, first)
    vs = _lif_step(vv_ref, v_in, first)

    rmat = r_ref[0]
    k_agg = jnp.dot(rmat, ks.reshape(NW, WIN * C),
                    preferred_element_type=jnp.float32).reshape(NW, WIN, C)
    v_agg = jnp.dot(rmat, vs.reshape(NW, WIN * C),
                    preferred_element_type=jnp.float32).reshape(NW, WIN, C)

    prod = (qs * k_agg).reshape(NW * WIN, C)
    attn_pre = jnp.dot(prod, seg_ref[...],
                       preferred_element_type=jnp.float32)
    a_s = _lif_step(va_ref, attn_pre, first)
    a_exp = jnp.dot(a_s, seg_ref[...].T,
                    preferred_element_type=jnp.float32).reshape(NW, WIN, C)

    o_in = a_exp * v_agg
    os_ = _lif_step(vo_ref, o_in, first)

    y = jnp.dot(os_.reshape(NW * WIN, C), wp_ref[...],
                preferred_element_type=jnp.float32)
    yw_ref[0, 0] = y.reshape(NW, LD, LH * LW, C)
    st_ref[0, 0, 0, :] = jnp.sum(y, axis=0)
    st_ref[0, 0, 1, :] = jnp.sum(y * y, axis=0)


def _run_k3(qw, kw, xw, r, aff, wp, seg):
    full = pl.BlockSpec((1, 1, NW, LD, LH * LW, C),
                        lambda b, t: (t, b, 0, 0, 0, 0))
    return pl.pallas_call(
        _k3_body,
        grid=(B, T),
        in_specs=[full, full, full,
                  pl.BlockSpec((1, NW, NW), lambda b, t: (b, 0, 0)),
                  pl.BlockSpec((6, C), lambda b, t: (0, 0)),
                  pl.BlockSpec((C, C), lambda b, t: (0, 0)),
                  pl.BlockSpec((C, HEADS), lambda b, t: (0, 0))],
        out_specs=[full,
                   pl.BlockSpec((1, 1, 2, C), lambda b, t: (b, t, 0, 0))],
        out_shape=(jax.ShapeDtypeStruct((T, B, NW, LD, LH * LW, C),
                                        jnp.float32),
                   jax.ShapeDtypeStruct((B, T, 2, C), jnp.float32)),
        scratch_shapes=[
            pltpu.VMEM((NW, WIN, C), jnp.float32),
            pltpu.VMEM((NW, WIN, C), jnp.float32),
            pltpu.VMEM((NW, WIN, C), jnp.float32),
            pltpu.VMEM((NW * WIN, HEADS), jnp.float32),
            pltpu.VMEM((NW, WIN, C), jnp.float32),
        ],
    )(qw, kw, xw, r, aff, wp, seg)


def _k4_body(yw_ref, aff_ref, out_ref):
    out_ref[0, 0] = yw_ref[0, 0] * aff_ref[0, :] + aff_ref[1, :]


def _run_k4(yw, aff):
    spec = pl.BlockSpec((1, 1, NW, LD, LH * LW, C),
                        lambda n: (n // B, n % B, 0, 0, 0, 0))
    return pl.pallas_call(
        _k4_body,
        grid=(N,),
        in_specs=[spec, pl.BlockSpec((2, C), lambda n: (0, 0))],
        out_specs=spec,
        out_shape=jax.ShapeDtypeStruct((T, B, NW, LD, LH * LW, C),
                                       jnp.float32),
    )(yw, aff)


def _affine(g, b, ssum, ssq, count):
    mean = ssum / count
    var = ssq / count - mean * mean
    scale = g * jax.lax.rsqrt(var + EPS)
    return scale, b - mean * scale


@jax.jit
def kernel(x, Wq, gq, bq, Wk, gk, bk, gv, bv, Wp, gp, bp):
    kd = Wq[:, :, 0].sum(axis=(2, 3)) + Wq[:, :, 2].sum(axis=(2, 3))
    w_eff = Wq.at[:, :, 1, 1, 1].add(-THETA * kd)
    wq_t = jnp.transpose(w_eff, (2, 3, 4, 1, 0)).reshape(27, C, C)
    wk_t = Wk[:, :, 0, 0, 0].T
    wp_t = Wp[:, :, 0, 0, 0].T
    seg = (jax.lax.broadcasted_iota(jnp.int32, (C, HEADS), 0) // (C // HEADS)
           == jax.lax.broadcasted_iota(jnp.int32, (C, HEADS), 1)
           ).astype(jnp.float32)

    xl = jnp.transpose(x.reshape(N, C, D, H, W), (0, 2, 3, 4, 1))
    xp = jnp.pad(xl, ((0, 0), (1, 1), (1, 1), (1, 1), (0, 0)))

    qw, kw, xw, st, reg = _run_k1(xp, wq_t, wk_t)

    stt = st.sum(axis=(0, 1))
    cnt = float(N * S)
    qsc, qsh = _affine(gq, bq, stt[0], stt[1], cnt)
    ksc, ksh = _affine(gk, bk, stt[2], stt[3], cnt)
    vsc, vsh = _affine(gv, bv, stt[4], stt[5], cnt)
    aff2 = jnp.stack([qsc, qsh, ksc, ksh])
    aff6 = jnp.stack([qsc, qsh, ksc, ksh, vsc, vsh])

    regb = reg.reshape(T, B, 2, LD, 2, 4, C)
    regs = regb.sum(axis=(0, 3))
    regs = jnp.transpose(regs, (0, 2, 1, 3, 4)).reshape(B, 2, NW, C)
    r = _run_k2(regs, aff2)

    yw, st3 = _run_k3(qw, kw, xw, r, aff6, wp_t, seg)
    stt3 = st3.sum(axis=(0, 1))
    psc, psh = _affine(gp, bp, stt3[0], stt3[1], cnt)
    outw = _run_k4(yw, jnp.stack([psc, psh]))

    o = outw.reshape(T, B, 2, 2, 2, LD, LH, LW, C)
    o = jnp.transpose(o, (0, 1, 8, 2, 5, 3, 6, 4, 7))
    return o.reshape(T, B, C, D, H, W)


# speedup vs baseline: 1.1729x; 1.0058x over previous
import jax
import jax.numpy as jnp
from jax.experimental import pallas as pl
from jax.experimental.pallas import tpu as pltpu

HEADS = 4
THETA = 0.7
VTH = 1.0
TOPK = 4
TAU = 2.0

T, B, C, D, H, W = 4, 2, 96, 8, 28, 28
N = T * B
NW = 8
LD, LH, LW = 4, 14, 14
WIN = LD * LH * LW
S = D * H * W
EPS = 1e-5


def _lif_step(vref, x, first):
    v_prev = jnp.where(first, jnp.zeros_like(x), vref[...])
    v = v_prev + (x - v_prev) / TAU
    s = (v >= VTH).astype(jnp.float32)
    vref[...] = v * (1.0 - s)
    return s


def _k1_body(xp_ref, wq_ref, wk_ref, qw_ref, kw_ref, xw_ref, st_ref, reg_ref):
    d = pl.program_id(1)

    def win_split(t):
        t = t.reshape(2, LH, 2, LW, C)
        t = jnp.transpose(t, (0, 2, 1, 3, 4))
        return t.reshape(4, LH * LW, C)

    acc = jnp.zeros((H * W, C), jnp.float32)
    for tap in range(27):
        dz, rem = tap // 9, tap % 9
        dy, dx = rem // 3, rem % 3
        plane = xp_ref[0, d + dz, dy:dy + H, dx:dx + W, :].reshape(H * W, C)
        acc = acc + jnp.dot(plane, wq_ref[tap],
                            preferred_element_type=jnp.float32)
    center = xp_ref[0, d + 1, 1:1 + H, 1:1 + W, :].reshape(H * W, C)
    kout = jnp.dot(center, wk_ref[...], preferred_element_type=jnp.float32)

    qsplit = win_split(acc.reshape(H, W, C))
    ksplit = win_split(kout.reshape(H, W, C))
    xsplit = win_split(center.reshape(H, W, C))
    qw_ref[0, 0, :, 0, :, :] = qsplit
    kw_ref[0, 0, :, 0, :, :] = ksplit
    xw_ref[0, 0, :, 0, :, :] = xsplit

    st_ref[0, 0, 0, :] = jnp.sum(acc, axis=0)
    st_ref[0, 0, 1, :] = jnp.sum(acc * acc, axis=0)
    st_ref[0, 0, 2, :] = jnp.sum(kout, axis=0)
    st_ref[0, 0, 3, :] = jnp.sum(kout * kout, axis=0)
    st_ref[0, 0, 4, :] = jnp.sum(center, axis=0)
    st_ref[0, 0, 5, :] = jnp.sum(center * center, axis=0)

    reg_ref[0, 0, 0:4, :] = jnp.sum(qsplit, axis=1)
    reg_ref[0, 0, 4:8, :] = jnp.sum(ksplit, axis=1)


def _run_k1(xp, wq, wk):
    grid = (N, D)
    out_shapes = (
        jax.ShapeDtypeStruct((T, B, NW, LD, LH * LW, C), jnp.float32),
        jax.ShapeDtypeStruct((T, B, NW, LD, LH * LW, C), jnp.float32),
        jax.ShapeDtypeStruct((T, B, NW, LD, LH * LW, C), jnp.float32),
        jax.ShapeDtypeStruct((N, D, 6, C), jnp.float32),
        jax.ShapeDtypeStruct((N, D, 8, C), jnp.float32),
    )
    win_spec = pl.BlockSpec(
        (1, 1, 4, 1, LH * LW, C),
        lambda n, d: (n // B, n % B, d // LD, d % LD, 0, 0),
    )
    return pl.pallas_call(
        _k1_body,
        grid=grid,
        in_specs=[
            pl.BlockSpec((1, D + 2, H + 2, W + 2, C), lambda n, d: (n, 0, 0, 0, 0)),
            pl.BlockSpec((27, C, C), lambda n, d: (0, 0, 0)),
            pl.BlockSpec((C, C), lambda n, d: (0, 0)),
        ],
        out_specs=[win_spec, win_spec, win_spec,
                   pl.BlockSpec((1, 1, 6, C), lambda n, d: (n, d, 0, 0)),
                   pl.BlockSpec((1, 1, 8, C), lambda n, d: (n, d, 0, 0))],
        out_shape=out_shapes,
    )(xp, wq, wk)


def _k3_body(qw_ref, kw_ref, xw_ref, reg_ref, aff_ref, wp_ref, seg_ref,
             yw_ref, st_ref, vq_ref, vk_ref, vv_ref, va_ref, vo_ref,
             rs_ref):
    t = pl.program_id(1)
    first = t == 0

    @pl.when(first)
    def _compute_routing():
        inv = 1.0 / (T * WIN)
        qreg = reg_ref[0, 0, :, :] * inv * aff_ref[0, :] + aff_ref[1, :]
        kreg = reg_ref[0, 1, :, :] * inv * aff_ref[2, :] + aff_ref[3, :]
        a = jnp.dot(qreg, kreg.T, preferred_element_type=jnp.float32)
        col = jax.lax.broadcasted_iota(jnp.int32, (NW, NW), 1)
        elig = jnp.ones((NW, NW), jnp.bool_)
        sel = jnp.zeros((NW, NW), jnp.float32)
        for _ in range(TOPK):
            masked = jnp.where(elig, a, -jnp.inf)
            m = jnp.max(masked, axis=1, keepdims=True)
            ism = masked >= m
            first_col = jnp.min(jnp.where(ism, col, NW), axis=1, keepdims=True)
            pick = col == first_col
            sel = sel + jnp.where(pick, 1.0 / TOPK, 0.0)
            elig = jnp.logical_and(elig, jnp.logical_not(pick))
        rs_ref[...] = sel

    q_in = qw_ref[0, 0].reshape(NW, WIN, C) * aff_ref[0, :] + aff_ref[1, :]
    k_in = kw_ref[0, 0].reshape(NW, WIN, C) * aff_ref[2, :] + aff_ref[3, :]
    v_in = xw_ref[0, 0].reshape(NW, WIN, C) * aff_ref[4, :] + aff_ref[5, :]

    qs = _lif_step(vq_ref, q_in, first)
    ks = _lif_step(vk_ref, k_in, first)
    vs = _lif_step(vv_ref, v_in, first)

    rmat = rs_ref[...]
    k_agg = jnp.dot(rmat, ks.reshape(NW, WIN * C),
                    preferred_element_type=jnp.float32).reshape(NW, WIN, C)
    v_agg = jnp.dot(rmat, vs.reshape(NW, WIN * C),
                    preferred_element_type=jnp.float32).reshape(NW, WIN, C)

    prod = (qs * k_agg).reshape(NW * WIN, C)
    attn_pre = jnp.dot(prod, seg_ref[...],
                       preferred_element_type=jnp.float32)
    a_s = _lif_step(va_ref, attn_pre, first)
    a_exp = jnp.dot(a_s, seg_ref[...].T,
                    preferred_element_type=jnp.float32).reshape(NW, WIN, C)

    o_in = a_exp * v_agg
    os_ = _lif_step(vo_ref, o_in, first)

    y = jnp.dot(os_.reshape(NW * WIN, C), wp_ref[...],
                preferred_element_type=jnp.float32)
    yw_ref[0, 0] = y.reshape(NW, LD, LH * LW, C)
    st_ref[0, 0, 0, :] = jnp.sum(y, axis=0)
    st_ref[0, 0, 1, :] = jnp.sum(y * y, axis=0)


def _run_k3(qw, kw, xw, regs, aff, wp, seg):
    full = pl.BlockSpec((1, 1, NW, LD, LH * LW, C),
                        lambda b, t: (t, b, 0, 0, 0, 0))
    return pl.pallas_call(
        _k3_body,
        grid=(B, T),
        in_specs=[full, full, full,
                  pl.BlockSpec((1, 2, NW, C), lambda b, t: (b, 0, 0, 0)),
                  pl.BlockSpec((6, C), lambda b, t: (0, 0)),
                  pl.BlockSpec((C, C), lambda b, t: (0, 0)),
                  pl.BlockSpec((C, HEADS), lambda b, t: (0, 0))],
        out_specs=[full,
                   pl.BlockSpec((1, 1, 2, C), lambda b, t: (b, t, 0, 0))],
        out_shape=(jax.ShapeDtypeStruct((T, B, NW, LD, LH * LW, C),
                                        jnp.float32),
                   jax.ShapeDtypeStruct((B, T, 2, C), jnp.float32)),
        scratch_shapes=[
            pltpu.VMEM((NW, WIN, C), jnp.float32),
            pltpu.VMEM((NW, WIN, C), jnp.float32),
            pltpu.VMEM((NW, WIN, C), jnp.float32),
            pltpu.VMEM((NW * WIN, HEADS), jnp.float32),
            pltpu.VMEM((NW, WIN, C), jnp.float32),
            pltpu.VMEM((NW, NW), jnp.float32),
        ],
    )(qw, kw, xw, regs, aff, wp, seg)


def _k4_body(yw_ref, aff_ref, out_ref):
    out_ref[0, 0] = yw_ref[0, 0] * aff_ref[0, :] + aff_ref[1, :]


def _run_k4(yw, aff):
    spec = pl.BlockSpec((1, 1, NW, LD, LH * LW, C),
                        lambda n: (n // B, n % B, 0, 0, 0, 0))
    return pl.pallas_call(
        _k4_body,
        grid=(N,),
        in_specs=[spec, pl.BlockSpec((2, C), lambda n: (0, 0))],
        out_specs=spec,
        out_shape=jax.ShapeDtypeStruct((T, B, NW, LD, LH * LW, C),
                                       jnp.float32),
    )(yw, aff)


def _affine(g, b, ssum, ssq, count):
    mean = ssum / count
    var = ssq / count - mean * mean
    scale = g * jax.lax.rsqrt(var + EPS)
    return scale, b - mean * scale


@jax.jit
def kernel(x, Wq, gq, bq, Wk, gk, bk, gv, bv, Wp, gp, bp):
    kd = Wq[:, :, 0].sum(axis=(2, 3)) + Wq[:, :, 2].sum(axis=(2, 3))
    w_eff = Wq.at[:, :, 1, 1, 1].add(-THETA * kd)
    wq_t = jnp.transpose(w_eff, (2, 3, 4, 1, 0)).reshape(27, C, C)
    wk_t = Wk[:, :, 0, 0, 0].T
    wp_t = Wp[:, :, 0, 0, 0].T
    seg = (jax.lax.broadcasted_iota(jnp.int32, (C, HEADS), 0) // (C // HEADS)
           == jax.lax.broadcasted_iota(jnp.int32, (C, HEADS), 1)
           ).astype(jnp.float32)

    xl = jnp.transpose(x.reshape(N, C, D, H, W), (0, 2, 3, 4, 1))
    xp = jnp.pad(xl, ((0, 0), (1, 1), (1, 1), (1, 1), (0, 0)))

    qw, kw, xw, st, reg = _run_k1(xp, wq_t, wk_t)

    stt = st.sum(axis=(0, 1))
    cnt = float(N * S)
    qsc, qsh = _affine(gq, bq, stt[0], stt[1], cnt)
    ksc, ksh = _affine(gk, bk, stt[2], stt[3], cnt)
    vsc, vsh = _affine(gv, bv, stt[4], stt[5], cnt)
    aff6 = jnp.stack([qsc, qsh, ksc, ksh, vsc, vsh])

    regb = reg.reshape(T, B, 2, LD, 2, 4, C)
    regs = regb.sum(axis=(0, 3))
    regs = jnp.transpose(regs, (0, 2, 1, 3, 4)).reshape(B, 2, NW, C)

    yw, st3 = _run_k3(qw, kw, xw, regs, aff6, wp_t, seg)
    stt3 = st3.sum(axis=(0, 1))
    psc, psh = _affine(gp, bp, stt3[0], stt3[1], cnt)
    outw = _run_k4(yw, jnp.stack([psc, psh]))

    o = outw.reshape(T, B, 2, 2, 2, LD, LH, LW, C)
    o = jnp.transpose(o, (0, 1, 8, 2, 5, 3, 6, 4, 7))
    return o.reshape(T, B, C, D, H, W)
